# Initial kernel scaffold; baseline (speedup 1.0000x reference)
import jax
import jax.numpy as jnp
from jax.experimental import pallas as pl


def kernel(x, edge_index, edge_attr, params):
    raise NotImplementedError("write your pallas kernel here")


# speedup vs baseline: 2.8590x; 2.8590x over previous
import functools

import jax
import jax.numpy as jnp
from jax import lax
from jax.experimental import pallas as pl
from jax.experimental.pallas import tpu as pltpu
from jax.experimental.pallas import tpu_sc as plsc

N = 10000
E = 320000
F = 8
NC, NS = 2, 16
NW = NC * NS
EPW = E // NW
CHUNK = 2000
NPS = N // NS

_f32 = jnp.float32


def _sc_mesh():
    return plsc.VectorSubcoreMesh(
        core_axis_name="c", subcore_axis_name="s",
        num_cores=NC, num_subcores=NS)


def _gather_body(x_hbm, src_hbm, dst_hbm, hd_out, hs_out, idx_v, rows_v, sem):
    c = lax.axis_index("c")
    s = lax.axis_index("s")
    wid = s * NC + c
    base = wid * EPW

    def step(i, carry):
        off = base + i * CHUNK
        pltpu.sync_copy(dst_hbm.at[pl.ds(off, CHUNK)], idx_v)
        pltpu.async_copy(x_hbm.at[idx_v], rows_v, sem).wait()
        pltpu.sync_copy(rows_v, hd_out.at[pl.ds(off, CHUNK)])
        pltpu.sync_copy(src_hbm.at[pl.ds(off, CHUNK)], idx_v)
        pltpu.async_copy(x_hbm.at[idx_v], rows_v, sem).wait()
        pltpu.sync_copy(rows_v, hs_out.at[pl.ds(off, CHUNK)])
        return carry

    lax.fori_loop(0, EPW // CHUNK, step, 0)


def _sc_gather(x8, src, dst):
    return pl.kernel(
        _gather_body,
        out_type=(jax.ShapeDtypeStruct((E, F), _f32),
                  jax.ShapeDtypeStruct((E, F), _f32)),
        mesh=_sc_mesh(),
        scratch_types=[
            pltpu.VMEM((CHUNK,), jnp.int32),
            pltpu.VMEM((CHUNK, F), _f32),
            pltpu.SemaphoreType.DMA,
        ],
        compiler_params=pltpu.CompilerParams(use_tc_tiling_on_sc=False),
    )(x8, src, dst)


def _scatter_body(m_hbm, dst_hbm, zeros_hbm, out_hbm,
                  idx_v, vals_v, zbuf_v, agg_sh):
    c = lax.axis_index("c")
    s = lax.axis_index("s")
    wid = s * NC + c
    pltpu.sync_copy(zeros_hbm.at[pl.ds(s * NPS, NPS)], zbuf_v)
    pltpu.sync_copy(zbuf_v, agg_sh.at[pl.ds(s * NPS, NPS)])
    plsc.subcore_barrier()
    base = wid * EPW

    def step(i, carry):
        off = base + i * CHUNK
        pltpu.sync_copy(dst_hbm.at[pl.ds(off, CHUNK)], idx_v)
        pltpu.sync_copy(m_hbm.at[pl.ds(off, CHUNK)], vals_v)
        pltpu.sync_copy(vals_v, agg_sh.at[idx_v], add=True)
        return carry

    lax.fori_loop(0, EPW // CHUNK, step, 0)
    plsc.subcore_barrier()
    pltpu.sync_copy(agg_sh.at[pl.ds(s * NPS, NPS)],
                    out_hbm.at[c, pl.ds(s * NPS, NPS)])


def _sc_scatter(m, dst, zeros_n):
    return pl.kernel(
        _scatter_body,
        out_type=jax.ShapeDtypeStruct((NC, N, F), _f32),
        mesh=_sc_mesh(),
        scratch_types=[
            pltpu.VMEM((CHUNK,), jnp.int32),
            pltpu.VMEM((CHUNK, F), _f32),
            pltpu.VMEM((NPS, F), _f32),
            pltpu.VMEM_SHARED((N, F), _f32),
        ],
        compiler_params=pltpu.CompilerParams(use_tc_tiling_on_sc=False),
    )(m, dst, zeros_n)


def _edge_mlp_body(nin, sigmoid_out, *refs):
    ins = refs[:nin]
    w1s = refs[nin:2 * nin]
    b1, w2, b2, out = refs[2 * nin:2 * nin + 4]
    acc = jnp.dot(ins[0][...], w1s[0][...], preferred_element_type=_f32)
    for k in range(1, nin):
        acc = acc + jnp.dot(ins[k][...], w1s[k][...],
                            preferred_element_type=_f32)
    h = jnp.maximum(acc + b1[...], 0.0)
    o = jnp.dot(h, w2[...], preferred_element_type=_f32) + b2[...]
    out[...] = jax.nn.sigmoid(o) if sigmoid_out else o


def _edge_mlp(ins, w1s, b1, w2, b2, sigmoid_out=False, block=2000):
    nin = len(ins)
    in_specs = [pl.BlockSpec((block, a.shape[1]), lambda i: (i, 0))
                for a in ins]
    in_specs += [pl.BlockSpec(w.shape, lambda i: (0, 0))
                 for w in (*w1s, b1, w2, b2)]
    dout = w2.shape[1]
    return pl.pallas_call(
        functools.partial(_edge_mlp_body, nin, sigmoid_out),
        grid=(E // block,),
        in_specs=in_specs,
        out_specs=pl.BlockSpec((block, dout), lambda i: (i, 0)),
        out_shape=jax.ShapeDtypeStruct((E, dout), _f32),
    )(*ins, *w1s, b1, w2, b2)


def _node_mlp_body(x_ref, agg_ref, hb_ref, w1x, w1a, b1, w2, b2, out_ref):
    a = agg_ref[0] + agg_ref[1]
    acc = jnp.dot(x_ref[...], w1x[...], preferred_element_type=_f32)
    acc = acc + jnp.dot(a, w1a[...], preferred_element_type=_f32)
    h = jnp.maximum(acc + b1[...], 0.0)
    out_ref[...] = (hb_ref[...]
                    + jnp.dot(h, w2[...], preferred_element_type=_f32)
                    + b2[...])


def _node_mlp(xk, agg2, h8, w1x, w1a, b1, w2, b2, block=2000):
    in_specs = [
        pl.BlockSpec((block, F), lambda i: (i, 0)),
        pl.BlockSpec((NC, block, F), lambda i: (0, i, 0)),
        pl.BlockSpec((block, F), lambda i: (i, 0)),
    ]
    in_specs += [pl.BlockSpec(w.shape, lambda i: (0, 0))
                 for w in (w1x, w1a, b1, w2, b2)]
    return pl.pallas_call(
        _node_mlp_body,
        grid=(N // block,),
        in_specs=in_specs,
        out_specs=pl.BlockSpec((block, F), lambda i: (i, 0)),
        out_shape=jax.ShapeDtypeStruct((N, F), _f32),
    )(xk, agg2, h8, w1x, w1a, b1, w2, b2)


def _enc_body(x_ref, w_ref, b_ref, out_ref):
    out_ref[...] = (jnp.dot(x_ref[...], w_ref[...],
                            preferred_element_type=_f32) + b_ref[...])


def _encoder(x, w, b, block=2000):
    return pl.pallas_call(
        _enc_body,
        grid=(N // block,),
        in_specs=[
            pl.BlockSpec((block, x.shape[1]), lambda i: (i, 0)),
            pl.BlockSpec(w.shape, lambda i: (0, 0)),
            pl.BlockSpec(b.shape, lambda i: (0, 0)),
        ],
        out_specs=pl.BlockSpec((block, F), lambda i: (i, 0)),
        out_shape=jax.ShapeDtypeStruct((N, F), _f32),
    )(x, w, b)


def _final_body(x_ref, bw1, bb1, bw2, bb2, xw1, xb1, xw2, xb2,
                beta_ref, hc_ref):
    xv = x_ref[...]
    hb = jnp.maximum(jnp.dot(xv, bw1[...], preferred_element_type=_f32)
                     + bb1[...], 0.0)
    beta_ref[...] = jax.nn.sigmoid(
        jnp.dot(hb, bw2[...], preferred_element_type=_f32) + bb2[...])
    hx = jnp.maximum(jnp.dot(xv, xw1[...], preferred_element_type=_f32)
                     + xb1[...], 0.0)
    hc_ref[...] = jnp.dot(hx, xw2[...], preferred_element_type=_f32) + xb2[...]


def _final(x7, weights, dout_hc, block=2000):
    in_specs = [pl.BlockSpec((block, F), lambda i: (i, 0))]
    in_specs += [pl.BlockSpec(w.shape, lambda i: (0, 0)) for w in weights]
    return pl.pallas_call(
        _final_body,
        grid=(N // block,),
        in_specs=in_specs,
        out_specs=[pl.BlockSpec((block, 1), lambda i: (i, 0)),
                   pl.BlockSpec((block, dout_hc), lambda i: (i, 0))],
        out_shape=[jax.ShapeDtypeStruct((N, 1), _f32),
                   jax.ShapeDtypeStruct((N, dout_hc), _f32)],
    )(x7, *weights)


def _pad_rows(w, rows):
    return jnp.pad(w, ((0, rows - w.shape[0]), (0, 0)))


def _pad_cols(w, cols):
    return jnp.pad(w, ((0, 0), (0, cols - w.shape[1])))


def _brow(b, cols=None):
    b2 = b[None, :]
    if cols is not None:
        b2 = _pad_cols(b2, cols)
    return b2


def kernel(x, edge_index, edge_attr, params):
    src = edge_index[0]
    dst = edge_index[1]
    zeros_n = jnp.zeros((N, F), _f32)

    h8 = _encoder(x, _pad_cols(params["enc_W"], F),
                  _brow(params["enc_b"], F))

    def layer(p, xk, attrs):
        hd, hs = _sc_gather(xk, src, dst)
        e_w1 = p["edge"]["W1"]
        ins = [hd, hs]
        segs = [_pad_rows(e_w1[0:7], F), _pad_rows(e_w1[7:14], F)]
        r = 14
        for arr, w in attrs:
            ins.append(arr)
            segs.append(_pad_rows(e_w1[r:r + w], arr.shape[1]))
            r += w
        m = _edge_mlp(ins, segs, _brow(p["edge"]["b1"]),
                      _pad_cols(p["edge"]["W2"], F),
                      _brow(p["edge"]["b2"], F))
        agg2 = _sc_scatter(m, dst, zeros_n)
        n_w1 = p["node"]["W1"]
        eout = p["edge"]["W2"].shape[1]
        xn = _node_mlp(xk, agg2, h8,
                       _pad_rows(n_w1[0:7], F),
                       _pad_rows(n_w1[7:7 + eout], F),
                       _brow(p["node"]["b1"]),
                       _pad_cols(p["node"]["W2"], F),
                       _brow(p["node"]["b2"], F))
        return xn, m

    x2, e1 = layer(params["in_w1"], h8, [(edge_attr, 4)])
    x3, e2 = layer(params["in_w2"], x2, [(e1, 4)])
    x4, e3 = layer(params["in_w3"], x3, [(e2, 4)])

    w_w1 = params["W"]["W1"]
    ew = _edge_mlp(
        [edge_attr, e1, e2, e3],
        [w_w1[0:4], _pad_rows(w_w1[4:8], F),
         _pad_rows(w_w1[8:12], F), _pad_rows(w_w1[12:16], F)],
        _brow(params["W"]["b1"]), params["W"]["W2"],
        _brow(params["W"]["b2"]), sigmoid_out=True)

    x5, ec1 = layer(params["in_c1"], x4,
                    [(ew, 1), (edge_attr, 4), (e1, 4), (e2, 4), (e3, 4)])
    x6, ec2 = layer(params["in_c2"], x5, [(ec1, 8)])
    x7, _ec3 = layer(params["in_c3"], x6, [(ec2, 8)])

    p_b, p_x = params["B"], params["X"]
    beta, hc = _final(
        x7,
        [_pad_rows(p_b["W1"], F), _brow(p_b["b1"]),
         p_b["W2"], _brow(p_b["b2"]),
         _pad_rows(p_x["W1"], F), _brow(p_x["b1"]),
         p_x["W2"], _brow(p_x["b2"])],
        p_x["W2"].shape[1])
    return (ew, hc, beta)


# speedup vs baseline: 12.8806x; 4.5053x over previous
import functools

import jax
import jax.numpy as jnp
from jax import lax
from jax.experimental import pallas as pl
from jax.experimental.pallas import tpu as pltpu
from jax.experimental.pallas import tpu_sc as plsc

N = 10000
E = 320000
F = 8
P = 16
EP = E // P
NP = N // P
NC, NS = 2, 16
NW = NC * NS
EPW = E // NW
NPS = N // NS

_f32 = jnp.float32


def _sc_mesh():
    return plsc.VectorSubcoreMesh(
        core_axis_name="c", subcore_axis_name="s",
        num_cores=NC, num_subcores=NS)


def _gather_body(x_hbm, ei_hbm, hd_out, hs_out, idx_v, rows_v, sem):
    c = lax.axis_index("c")
    s = lax.axis_index("s")
    wid = s * NC + c
    base = wid * EPW
    pltpu.sync_copy(ei_hbm.at[pl.ds(E + base, EPW)], idx_v)
    pltpu.async_copy(x_hbm.at[idx_v], rows_v, sem).wait()
    pltpu.sync_copy(rows_v, hd_out.at[pl.ds(base, EPW)])
    pltpu.sync_copy(ei_hbm.at[pl.ds(base, EPW)], idx_v)
    pltpu.async_copy(x_hbm.at[idx_v], rows_v, sem).wait()
    pltpu.sync_copy(rows_v, hs_out.at[pl.ds(base, EPW)])


def _sc_gather(x8, ei_flat):
    return pl.kernel(
        _gather_body,
        out_type=(jax.ShapeDtypeStruct((E, F), _f32),
                  jax.ShapeDtypeStruct((E, F), _f32)),
        mesh=_sc_mesh(),
        scratch_types=[
            pltpu.VMEM((EPW,), jnp.int32),
            pltpu.VMEM((EPW, F), _f32),
            pltpu.SemaphoreType.DMA,
        ],
        compiler_params=pltpu.CompilerParams(use_tc_tiling_on_sc=False),
    )(x8, ei_flat)


def _scatter_body(m_hbm, ei_hbm, zeros_hbm, out_hbm,
                  idx_v, vals_v, zbuf_v, agg_sh):
    c = lax.axis_index("c")
    s = lax.axis_index("s")
    wid = s * NC + c
    pltpu.sync_copy(zeros_hbm.at[pl.ds(s * NPS, NPS)], zbuf_v)
    pltpu.sync_copy(zbuf_v, agg_sh.at[pl.ds(s * NPS, NPS)])
    plsc.subcore_barrier()
    base = wid * EPW
    pltpu.sync_copy(ei_hbm.at[pl.ds(E + base, EPW)], idx_v)
    pltpu.sync_copy(m_hbm.at[pl.ds(base, EPW)], vals_v)
    pltpu.sync_copy(vals_v, agg_sh.at[idx_v], add=True)
    plsc.subcore_barrier()
    pltpu.sync_copy(agg_sh.at[pl.ds(s * NPS, NPS)],
                    out_hbm.at[c, pl.ds(s * NPS, NPS)])


def _sc_scatter(m, ei_flat, zeros_n):
    return pl.kernel(
        _scatter_body,
        out_type=jax.ShapeDtypeStruct((NC, N, F), _f32),
        mesh=_sc_mesh(),
        scratch_types=[
            pltpu.VMEM((EPW,), jnp.int32),
            pltpu.VMEM((EPW, F), _f32),
            pltpu.VMEM((NPS, F), _f32),
            pltpu.VMEM_SHARED((N, F), _f32),
        ],
        compiler_params=pltpu.CompilerParams(use_tc_tiling_on_sc=False),
    )(m, ei_flat, zeros_n)


def _mlp_body(nin, sigmoid_out, *refs):
    ins = refs[:nin]
    w1s = refs[nin:2 * nin]
    b1, w2, b2, out = refs[2 * nin:2 * nin + 4]
    acc = jnp.dot(ins[0][...], w1s[0][...], preferred_element_type=_f32)
    for k in range(1, nin):
        acc = acc + jnp.dot(ins[k][...], w1s[k][...],
                            preferred_element_type=_f32)
    h = jnp.maximum(acc + b1[...], 0.0)
    o = jnp.dot(h, w2[...], preferred_element_type=_f32) + b2[...]
    out[...] = jax.nn.sigmoid(o) if sigmoid_out else o


def _packed_mlp(ins, w1s, b1, w2, b2, sigmoid_out=False, block=1000):
    nrows = ins[0].shape[0]
    nin = len(ins)
    in_specs = [pl.BlockSpec((block, a.shape[1]), lambda i: (i, 0))
                for a in ins]
    in_specs += [pl.BlockSpec(w.shape, lambda i: (0, 0))
                 for w in (*w1s, b1, w2, b2)]
    dout = w2.shape[1]
    return pl.pallas_call(
        functools.partial(_mlp_body, nin, sigmoid_out),
        grid=(nrows // block,),
        in_specs=in_specs,
        out_specs=pl.BlockSpec((block, dout), lambda i: (i, 0)),
        out_shape=jax.ShapeDtypeStruct((nrows, dout), _f32),
    )(*ins, *w1s, b1, w2, b2)


def _enc_body(x_ref, w_ref, b_ref, out_ref):
    out_ref[...] = (jnp.dot(x_ref[...], w_ref[...],
                            preferred_element_type=_f32) + b_ref[...])


def _encoder(x_g, w, b, block=NP):
    return pl.pallas_call(
        _enc_body,
        grid=(NP // block,),
        in_specs=[
            pl.BlockSpec((block, x_g.shape[1]), lambda i: (i, 0)),
            pl.BlockSpec(w.shape, lambda i: (0, 0)),
            pl.BlockSpec(b.shape, lambda i: (0, 0)),
        ],
        out_specs=pl.BlockSpec((block, 128), lambda i: (i, 0)),
        out_shape=jax.ShapeDtypeStruct((NP, 128), _f32),
    )(x_g, w, b)


def _node_mlp_body(x_ref, agg_ref, hb_ref, w1x, w1a, b1, w2, b2, out_ref):
    a = agg_ref[0] + agg_ref[1]
    acc = jnp.dot(x_ref[...], w1x[...], preferred_element_type=_f32)
    acc = acc + jnp.dot(a, w1a[...], preferred_element_type=_f32)
    h = jnp.maximum(acc + b1[...], 0.0)
    out_ref[...] = (hb_ref[...]
                    + jnp.dot(h, w2[...], preferred_element_type=_f32)
                    + b2[...])


def _node_mlp(xk_p, agg_p, h_p, w1x, w1a, b1, w2, b2):
    in_specs = [
        pl.BlockSpec((NP, 128), lambda i: (i, 0)),
        pl.BlockSpec((NC, NP, 128), lambda i: (0, i, 0)),
        pl.BlockSpec((NP, 128), lambda i: (i, 0)),
    ]
    in_specs += [pl.BlockSpec(w.shape, lambda i: (0, 0))
                 for w in (w1x, w1a, b1, w2, b2)]
    return pl.pallas_call(
        _node_mlp_body,
        grid=(1,),
        in_specs=in_specs,
        out_specs=pl.BlockSpec((NP, 128), lambda i: (i, 0)),
        out_shape=jax.ShapeDtypeStruct((NP, 128), _f32),
    )(xk_p, agg_p, h_p, w1x, w1a, b1, w2, b2)


def _final_body(x_ref, bw1, bb1, bw2, bb2, xw1, xb1, xw2, xb2,
                beta_ref, hc_ref):
    xv = x_ref[...]
    hb = jnp.maximum(jnp.dot(xv, bw1[...], preferred_element_type=_f32)
                     + bb1[...], 0.0)
    beta_ref[...] = jax.nn.sigmoid(
        jnp.dot(hb, bw2[...], preferred_element_type=_f32) + bb2[...])
    hx = jnp.maximum(jnp.dot(xv, xw1[...], preferred_element_type=_f32)
                     + xb1[...], 0.0)
    hc_ref[...] = jnp.dot(hx, xw2[...], preferred_element_type=_f32) + xb2[...]


def _final(x7_p, weights, dout_hc):
    in_specs = [pl.BlockSpec((NP, 128), lambda i: (i, 0))]
    in_specs += [pl.BlockSpec(w.shape, lambda i: (0, 0)) for w in weights]
    return pl.pallas_call(
        _final_body,
        grid=(1,),
        in_specs=in_specs,
        out_specs=[pl.BlockSpec((NP, P), lambda i: (i, 0)),
                   pl.BlockSpec((NP, dout_hc * P), lambda i: (i, 0))],
        out_shape=[jax.ShapeDtypeStruct((NP, P), _f32),
                   jax.ShapeDtypeStruct((NP, dout_hc * P), _f32)],
    )(x7_p, *weights)


def _pad_rows(w, rows):
    return jnp.pad(w, ((0, rows - w.shape[0]), (0, 0)))


def _bd(w):
    return jnp.kron(jnp.eye(P, dtype=w.dtype), w)


def _btile(b):
    return jnp.tile(b, P)[None, :]


def kernel(x, edge_index, edge_attr, params):
    ei_flat = edge_index.reshape(2 * E)
    attr_p = edge_attr.reshape(EP, 4 * P)
    zeros_n = jnp.zeros((N, F), _f32)

    x_g = x.reshape(NP, P * x.shape[1])
    enc_w = _bd(jnp.pad(params["enc_W"], ((0, 0), (0, 1))))
    enc_b = _btile(jnp.pad(params["enc_b"], (0, 1)))
    h_p = _encoder(x_g, enc_w, enc_b)

    def layer(p, xk_p, attrs):
        hd, hs = _sc_gather(xk_p.reshape(N, F), ei_flat)
        e_w1 = p["edge"]["W1"]
        ins = [hd.reshape(EP, 128), hs.reshape(EP, 128)]
        segs = [_bd(_pad_rows(e_w1[0:7], F)), _bd(_pad_rows(e_w1[7:14], F))]
        r = 14
        for arr, w in attrs:
            ins.append(arr)
            segs.append(_bd(_pad_rows(e_w1[r:r + w], arr.shape[1] // P)))
            r += w
        m_p = _packed_mlp(ins, segs, _btile(p["edge"]["b1"]),
                          _bd(jnp.pad(p["edge"]["W2"],
                                      ((0, 0), (0, F - p["edge"]["W2"].shape[1])))),
                          _btile(jnp.pad(p["edge"]["b2"],
                                         (0, F - p["edge"]["b2"].shape[0]))))
        agg2 = _sc_scatter(m_p.reshape(E, F), ei_flat, zeros_n)
        n_w1 = p["node"]["W1"]
        eout = p["edge"]["W2"].shape[1]
        xn_p = _node_mlp(
            xk_p, agg2.reshape(NC, NP, 128), h_p,
            _bd(_pad_rows(n_w1[0:7], F)),
            _bd(_pad_rows(n_w1[7:7 + eout], F)),
            _btile(p["node"]["b1"]),
            _bd(jnp.pad(p["node"]["W2"], ((0, 0), (0, 1)))),
            _btile(jnp.pad(p["node"]["b2"], (0, 1))))
        return xn_p, m_p

    x2_p, e1 = layer(params["in_w1"], h_p, [(attr_p, 4)])
    x3_p, e2 = layer(params["in_w2"], x2_p, [(e1, 4)])
    x4_p, e3 = layer(params["in_w3"], x3_p, [(e2, 4)])

    w_w1 = params["W"]["W1"]
    ew_p = _packed_mlp(
        [attr_p, e1, e2, e3],
        [_bd(w_w1[0:4]), _bd(_pad_rows(w_w1[4:8], F)),
         _bd(_pad_rows(w_w1[8:12], F)), _bd(_pad_rows(w_w1[12:16], F))],
        _btile(params["W"]["b1"]), _bd(params["W"]["W2"]),
        _btile(params["W"]["b2"]), sigmoid_out=True)

    x5_p, ec1 = layer(params["in_c1"], x4_p,
                      [(ew_p, 1), (attr_p, 4), (e1, 4), (e2, 4), (e3, 4)])
    x6_p, ec2 = layer(params["in_c2"], x5_p, [(ec1, 8)])
    x7_p, _ec3 = layer(params["in_c3"], x6_p, [(ec2, 8)])

    p_b, p_x = params["B"], params["X"]
    dout_hc = p_x["W2"].shape[1]
    beta_p, hc_p = _final(
        x7_p,
        [_bd(_pad_rows(p_b["W1"], F)), _btile(p_b["b1"]),
         _bd(p_b["W2"]), _btile(p_b["b2"]),
         _bd(_pad_rows(p_x["W1"], F)), _btile(p_x["b1"]),
         _bd(p_x["W2"]), _btile(p_x["b2"])],
        dout_hc)
    return (ew_p.reshape(E, 1), hc_p.reshape(N, dout_hc), beta_p.reshape(N, 1))
